# baseline (speedup 1.0000x reference)
import functools

import jax
import jax.numpy as jnp
from jax import lax
from jax.experimental import pallas as pl
from jax.experimental.pallas import tpu as pltpu
from jax.experimental.pallas import tpu_sc as plsc

NCLS = 19
CPAD = 20
HC = 128
HF = 512
BROWS = 10
BAND = CPAD * BROWS * HC
NSUB = 16
ROWS_PER_TILE = HF // NSUB


def _cls_body(preds_ref, masks_ref, cls_ref):
    best = preds_ref[0, 0]
    besti = jnp.zeros(best.shape, jnp.int32)
    for k in range(1, NCLS):
        pk = preds_ref[0, k]
        m = pk > best
        best = jnp.where(m, pk, best)
        besti = jnp.where(m, k, besti)
    mask = masks_ref[0]
    cls_ref[0] = jnp.where(besti == mask, mask, NCLS).astype(jnp.int32)


def _compute_cls(preds, masks):
    b = preds.shape[0]
    rb = 64
    grid = (b, HF // rb)
    return pl.pallas_call(
        _cls_body,
        grid=grid,
        in_specs=[
            pl.BlockSpec((1, NCLS, rb, HF), lambda i, j: (i, 0, j, 0)),
            pl.BlockSpec((1, rb, HF), lambda i, j: (i, j, 0)),
        ],
        out_specs=pl.BlockSpec((1, rb, HF), lambda i, j: (i, j, 0)),
        out_shape=jax.ShapeDtypeStruct((b, HF, HF), jnp.int32),
    )(preds, masks)


def _sc_body(cls_hbm, out_hbm, cls_v, band_v):
    c = lax.axis_index("c")
    s = lax.axis_index("s")
    npix = ROWS_PER_TILE * HF
    pltpu.sync_copy(cls_hbm.at[c, pl.ds(s * npix, npix)], cls_v)

    zero16 = jnp.zeros((16,), jnp.float32)
    def _zero(i, carry):
        for u in range(16):
            band_v[pl.ds(i * 256 + u * 16, 16)] = zero16
        return carry
    lax.fori_loop(0, BAND // 256, _zero, 0)

    lane = lax.iota(jnp.int32, 16)
    lane4 = lane * 4
    band_base_row = 8 * s - 1
    wtab = (0.375, 0.125, 0.875, 0.625)

    colm = []
    for m in range(8):
        c0 = lane + (16 * m - 1)
        if m == 0:
            c0 = jnp.maximum(c0, 0)
        c1 = lane + 16 * m
        c2 = lane + (16 * m + 1)
        if m == 7:
            c2 = jnp.minimum(c2, HC - 1)
        colm.append((c0, c1, c2))
    m_lo = lane == 0
    m_lo_rest = lane != 0
    m_hi = lane == 15
    m_hi_rest = lane != 15

    def _row(i, carry):
        ih = s * ROWS_PER_TILE + i
        jh = ih // 4
        rh = ih % 4
        jha = jnp.clip(jh + jnp.where(rh < 2, -1, 0), 0, HC - 1)
        jhb = jnp.clip(jh + jnp.where(rh < 2, 0, 1), 0, HC - 1)
        wha = jnp.where(rh == 0, 0.375,
              jnp.where(rh == 1, 0.125,
              jnp.where(rh == 2, 0.875, 0.625)))
        whb = 1.0 - wha
        rowa = jnp.full((16,), (jha - band_base_row) * HC, jnp.int32)
        rowb = jnp.full((16,), (jhb - band_base_row) * HC, jnp.int32)
        wa = [jnp.full((16,), wha * x, jnp.float32) for x in wtab]
        wb = [jnp.full((16,), whb * x, jnp.float32) for x in wtab]
        pix0 = lane4 + i * HF
        for r in range(4):
            la = 0 if r < 2 else 1
            for m in range(8):
                c0, c1, c2 = colm[m]
                ra = (rowa + c0, rowa + c1, rowa + c2)
                rb = (rowb + c0, rowb + c1, rowb + c2)
                cls16 = plsc.load_gather(cls_v, [pix0 + (64 * m + r)])
                base = cls16 * (BROWS * HC)
                for rowv, wv in ((ra, wa), (rb, wb)):
                    for tap, wgt in ((la, wv[r]), (la + 1, wv[3 - r])):
                        idx = base + rowv[tap]
                        if m == 0 and tap == 0:
                            plsc.addupdate_scatter(band_v, [idx], wgt, mask=m_lo)
                            plsc.addupdate_scatter(band_v, [idx], wgt, mask=m_lo_rest)
                        elif m == 7 and tap == 2:
                            plsc.addupdate_scatter(band_v, [idx], wgt, mask=m_hi)
                            plsc.addupdate_scatter(band_v, [idx], wgt, mask=m_hi_rest)
                        else:
                            plsc.addupdate_scatter(band_v, [idx], wgt)
        return carry
    lax.fori_loop(0, ROWS_PER_TILE, _row, 0)

    pltpu.sync_copy(band_v, out_hbm.at[c, s])


def _scatter_bands(cls):
    b = cls.shape[0]
    mesh = plsc.VectorSubcoreMesh(
        core_axis_name="c", subcore_axis_name="s",
        num_cores=b, num_subcores=NSUB)
    fn = functools.partial(
        pl.kernel,
        out_type=jax.ShapeDtypeStruct((b, NSUB, BAND), jnp.float32),
        mesh=mesh,
        scratch_types=[
            pltpu.VMEM((ROWS_PER_TILE * HF,), jnp.int32),
            pltpu.VMEM((BAND,), jnp.float32),
        ],
        compiler_params=pltpu.CompilerParams(needs_layout_passes=False,
                                             disable_bounds_checks=True),
    )(_sc_body)
    return fn(cls)


def _proto_body(bands_ref, feats_ref, out_ref, w_ref):
    b = pl.program_id(0)
    nb = pl.num_programs(0)
    w_ref[...] = jnp.zeros((CPAD, HC * HC), jnp.float32)
    for s in range(NSUB):
        lo = (8 * s - 1) * HC
        a = max(lo, 0)
        e = min(lo + BROWS * HC, HC * HC)
        w_ref[:, a:e] = w_ref[:, a:e] + bands_ref[0, s][:, a - lo:e - lo]
    wf = w_ref[0:NCLS, :]
    sums = jnp.sum(wf, axis=1)
    num = lax.dot_general(wf, feats_ref[0], (((1,), (1,)), ((), ())),
                          preferred_element_type=jnp.float32)
    contrib = num / (sums + 1e-6)[:, None] * (1.0 / nb)

    @pl.when(b == 0)
    def _init():
        out_ref[...] = contrib

    @pl.when(b != 0)
    def _acc():
        out_ref[...] = out_ref[...] + contrib


def _prototypes(bands, feats2d):
    b, c = feats2d.shape[0], feats2d.shape[1]
    return pl.pallas_call(
        _proto_body,
        grid=(b,),
        in_specs=[
            pl.BlockSpec((1, NSUB, CPAD, BROWS * HC), lambda i: (i, 0, 0, 0)),
            pl.BlockSpec((1, c, HC * HC), lambda i: (i, 0, 0)),
        ],
        out_specs=pl.BlockSpec((NCLS, c), lambda i: (0, 0)),
        out_shape=jax.ShapeDtypeStruct((NCLS, c), jnp.float32),
        scratch_shapes=[pltpu.VMEM((CPAD, HC * HC), jnp.float32)],
    )(bands, feats2d)


def kernel(feats, preds, masks):
    b, c = feats.shape[0], feats.shape[1]
    masks = masks.astype(jnp.int32)
    cls = _compute_cls(preds, masks)
    bands = _scatter_bands(cls.reshape(b, HF * HF))
    bands = bands.reshape(b, NSUB, CPAD, BROWS * HC)
    feats2d = feats.reshape(b, c, HC * HC)
    return _prototypes(bands, feats2d)


# speedup vs baseline: 1.0739x; 1.0739x over previous
import functools

import jax
import jax.numpy as jnp
from jax import lax
from jax.experimental import pallas as pl
from jax.experimental.pallas import tpu as pltpu
from jax.experimental.pallas import tpu_sc as plsc

NCLS = 19
CPAD = 20
HC = 128
HF = 512
BROWS = 10
BAND = CPAD * BROWS * HC
NSUB = 16
ROWS_PER_TILE = HF // NSUB


def _cls_body(preds_ref, masks_ref, cls_ref):
    best = preds_ref[0, 0]
    besti = jnp.zeros(best.shape, jnp.int32)
    for k in range(1, NCLS):
        pk = preds_ref[0, k]
        m = pk > best
        best = jnp.where(m, pk, best)
        besti = jnp.where(m, k, besti)
    mask = masks_ref[0]
    cls_ref[0] = jnp.where(besti == mask, mask, NCLS).astype(jnp.int32)


def _compute_cls(preds, masks):
    b = preds.shape[0]
    rb = 64
    grid = (b, HF // rb)
    return pl.pallas_call(
        _cls_body,
        grid=grid,
        in_specs=[
            pl.BlockSpec((1, NCLS, rb, HF), lambda i, j: (i, 0, j, 0)),
            pl.BlockSpec((1, rb, HF), lambda i, j: (i, j, 0)),
        ],
        out_specs=pl.BlockSpec((1, rb, HF), lambda i, j: (i, j, 0)),
        out_shape=jax.ShapeDtypeStruct((b, HF, HF), jnp.int32),
    )(preds, masks)


def _sc_body(cls_hbm, out_hbm, cls_v, band_v):
    c = lax.axis_index("c")
    s = lax.axis_index("s")
    npix = ROWS_PER_TILE * HF
    pltpu.sync_copy(cls_hbm.at[c, pl.ds(s * npix, npix)], cls_v)

    zero16 = jnp.zeros((16,), jnp.float32)

    @plsc.parallel_loop(0, BAND // 256)
    def _zero(i):
        for u in range(16):
            band_v[pl.ds(i * 256 + u * 16, 16)] = zero16

    lane = lax.iota(jnp.int32, 16)
    lane4 = lane * 4
    band_base_row = 8 * s - 1
    wtab = (0.375, 0.125, 0.875, 0.625)

    colm = []
    for m in range(8):
        c0 = lane + (16 * m - 1)
        if m == 0:
            c0 = jnp.maximum(c0, 0)
        c1 = lane + 16 * m
        c2 = lane + (16 * m + 1)
        if m == 7:
            c2 = jnp.minimum(c2, HC - 1)
        colm.append((c0, c1, c2))
    m_lo = lane == 0
    m_lo_rest = lane != 0
    m_hi = lane == 15
    m_hi_rest = lane != 15

    @plsc.parallel_loop(0, ROWS_PER_TILE)
    def _row(i):
        ih = s * ROWS_PER_TILE + i
        jh = ih // 4
        rh = ih % 4
        jha = jnp.clip(jh + jnp.where(rh < 2, -1, 0), 0, HC - 1)
        jhb = jnp.clip(jh + jnp.where(rh < 2, 0, 1), 0, HC - 1)
        wha = jnp.where(rh == 0, 0.375,
              jnp.where(rh == 1, 0.125,
              jnp.where(rh == 2, 0.875, 0.625)))
        whb = 1.0 - wha
        rowa = jnp.full((16,), (jha - band_base_row) * HC, jnp.int32)
        rowb = jnp.full((16,), (jhb - band_base_row) * HC, jnp.int32)
        wa = [jnp.full((16,), wha * x, jnp.float32) for x in wtab]
        wb = [jnp.full((16,), whb * x, jnp.float32) for x in wtab]
        pix0 = lane4 + i * HF
        for m in range(8):
            c0, c1, c2 = colm[m]
            ra = (rowa + c0, rowa + c1, rowa + c2)
            rb = (rowb + c0, rowb + c1, rowb + c2)
            cls4 = [plsc.load_gather(cls_v, [pix0 + (64 * m + r)])
                    for r in range(4)]
            base4 = [cv * (BROWS * HC) for cv in cls4]
            for r in range(4):
                la = 0 if r < 2 else 1
                base = base4[r]
                for rowv, wv in ((ra, wa), (rb, wb)):
                    for tap, wgt in ((la, wv[r]), (la + 1, wv[3 - r])):
                        idx = base + rowv[tap]
                        if m == 0 and tap == 0:
                            plsc.addupdate_scatter(band_v, [idx], wgt, mask=m_lo)
                            plsc.addupdate_scatter(band_v, [idx], wgt, mask=m_lo_rest)
                        elif m == 7 and tap == 2:
                            plsc.addupdate_scatter(band_v, [idx], wgt, mask=m_hi)
                            plsc.addupdate_scatter(band_v, [idx], wgt, mask=m_hi_rest)
                        else:
                            plsc.addupdate_scatter(band_v, [idx], wgt)

    pltpu.sync_copy(band_v, out_hbm.at[c, s])


def _scatter_bands(cls):
    b = cls.shape[0]
    mesh = plsc.VectorSubcoreMesh(
        core_axis_name="c", subcore_axis_name="s",
        num_cores=b, num_subcores=NSUB)
    fn = functools.partial(
        pl.kernel,
        out_type=jax.ShapeDtypeStruct((b, NSUB, BAND), jnp.float32),
        mesh=mesh,
        scratch_types=[
            pltpu.VMEM((ROWS_PER_TILE * HF,), jnp.int32),
            pltpu.VMEM((BAND,), jnp.float32),
        ],
        compiler_params=pltpu.CompilerParams(needs_layout_passes=False,
                                             disable_bounds_checks=True),
    )(_sc_body)
    return fn(cls)


def _proto_body(bands_ref, feats_ref, out_ref, w_ref):
    b = pl.program_id(0)
    nb = pl.num_programs(0)
    w_ref[...] = jnp.zeros((CPAD, HC * HC), jnp.float32)
    for s in range(NSUB):
        lo = (8 * s - 1) * HC
        a = max(lo, 0)
        e = min(lo + BROWS * HC, HC * HC)
        w_ref[:, a:e] = w_ref[:, a:e] + bands_ref[0, s][:, a - lo:e - lo]
    wf = w_ref[0:NCLS, :]
    sums = jnp.sum(wf, axis=1)
    num = lax.dot_general(wf, feats_ref[0], (((1,), (1,)), ((), ())),
                          preferred_element_type=jnp.float32)
    contrib = num / (sums + 1e-6)[:, None] * (1.0 / nb)

    @pl.when(b == 0)
    def _init():
        out_ref[...] = contrib

    @pl.when(b != 0)
    def _acc():
        out_ref[...] = out_ref[...] + contrib


def _prototypes(bands, feats2d):
    b, c = feats2d.shape[0], feats2d.shape[1]
    return pl.pallas_call(
        _proto_body,
        grid=(b,),
        in_specs=[
            pl.BlockSpec((1, NSUB, CPAD, BROWS * HC), lambda i: (i, 0, 0, 0)),
            pl.BlockSpec((1, c, HC * HC), lambda i: (i, 0, 0)),
        ],
        out_specs=pl.BlockSpec((NCLS, c), lambda i: (0, 0)),
        out_shape=jax.ShapeDtypeStruct((NCLS, c), jnp.float32),
        scratch_shapes=[pltpu.VMEM((CPAD, HC * HC), jnp.float32)],
    )(bands, feats2d)


def kernel(feats, preds, masks):
    b, c = feats.shape[0], feats.shape[1]
    masks = masks.astype(jnp.int32)
    cls = _compute_cls(preds, masks)
    bands = _scatter_bands(cls.reshape(b, HF * HF))
    bands = bands.reshape(b, NSUB, CPAD, BROWS * HC)
    feats2d = feats.reshape(b, c, HC * HC)
    return _prototypes(bands, feats2d)


# speedup vs baseline: 1.0850x; 1.0103x over previous
import functools

import jax
import jax.numpy as jnp
from jax import lax
from jax.experimental import pallas as pl
from jax.experimental.pallas import tpu as pltpu
from jax.experimental.pallas import tpu_sc as plsc

NCLS = 19
CPAD = 20
HC = 128
HF = 512
BROWS = 10
BAND = CPAD * BROWS * HC
NSUB = 16
ROWS_PER_TILE = HF // NSUB


def _cls_body(preds_ref, masks_ref, cls_ref):
    best = preds_ref[0, 0]
    besti = jnp.zeros(best.shape, jnp.int32)
    for k in range(1, NCLS):
        pk = preds_ref[0, k]
        m = pk > best
        best = jnp.where(m, pk, best)
        besti = jnp.where(m, k, besti)
    mask = masks_ref[0]
    cls_ref[0] = jnp.where(besti == mask, mask, NCLS).astype(jnp.int32)


def _compute_cls(preds, masks):
    b = preds.shape[0]
    rb = 64
    grid = (b, HF // rb)
    return pl.pallas_call(
        _cls_body,
        grid=grid,
        in_specs=[
            pl.BlockSpec((1, NCLS, rb, HF), lambda i, j: (i, 0, j, 0)),
            pl.BlockSpec((1, rb, HF), lambda i, j: (i, j, 0)),
        ],
        out_specs=pl.BlockSpec((1, rb, HF), lambda i, j: (i, j, 0)),
        out_shape=jax.ShapeDtypeStruct((b, HF, HF), jnp.int32),
    )(preds, masks)


def _sc_body(cls_hbm, out_hbm, cls_v, band_v):
    c = lax.axis_index("c")
    s = lax.axis_index("s")
    npix = ROWS_PER_TILE * HF
    pltpu.sync_copy(cls_hbm.at[c, pl.ds(s * npix, npix)], cls_v)

    zero16 = jnp.zeros((16,), jnp.float32)

    @plsc.parallel_loop(0, BAND // 256)
    def _zero(i):
        for u in range(16):
            band_v[pl.ds(i * 256 + u * 16, 16)] = zero16

    lane = lax.iota(jnp.int32, 16)
    lane4 = lane * 4
    band_base_row = 8 * s - 1
    wtab = (0.375, 0.125, 0.875, 0.625)

    colm = []
    for m in range(8):
        c0 = lane + (16 * m - 1)
        if m == 0:
            c0 = jnp.maximum(c0, 0)
        c1 = lane + 16 * m
        c2 = lane + (16 * m + 1)
        if m == 7:
            c2 = jnp.minimum(c2, HC - 1)
        colm.append((c0, c1, c2))
    m_lo = lane == 0
    m_lo_rest = lane != 0
    m_hi = lane == 15
    m_hi_rest = lane != 15

    @plsc.parallel_loop(0, ROWS_PER_TILE, unroll=4)
    def _row(i):
        ih = s * ROWS_PER_TILE + i
        jh = ih // 4
        rh = ih % 4
        jha = jnp.clip(jh + jnp.where(rh < 2, -1, 0), 0, HC - 1)
        jhb = jnp.clip(jh + jnp.where(rh < 2, 0, 1), 0, HC - 1)
        wha = jnp.where(rh == 0, 0.375,
              jnp.where(rh == 1, 0.125,
              jnp.where(rh == 2, 0.875, 0.625)))
        whb = 1.0 - wha
        rowa = jnp.full((16,), (jha - band_base_row) * HC, jnp.int32)
        rowb = jnp.full((16,), (jhb - band_base_row) * HC, jnp.int32)
        wa = [jnp.full((16,), wha * x, jnp.float32) for x in wtab]
        wb = [jnp.full((16,), whb * x, jnp.float32) for x in wtab]
        pix0 = lane4 + i * HF
        for m in range(8):
            c0, c1, c2 = colm[m]
            ra = (rowa + c0, rowa + c1, rowa + c2)
            rb = (rowb + c0, rowb + c1, rowb + c2)
            cls4 = [plsc.load_gather(cls_v, [pix0 + (64 * m + r)])
                    for r in range(4)]
            base4 = [cv * (BROWS * HC) for cv in cls4]
            for r in range(4):
                la = 0 if r < 2 else 1
                base = base4[r]
                for rowv, wv in ((ra, wa), (rb, wb)):
                    for tap, wgt in ((la, wv[r]), (la + 1, wv[3 - r])):
                        idx = base + rowv[tap]
                        if m == 0 and tap == 0:
                            plsc.addupdate_scatter(band_v, [idx], wgt, mask=m_lo)
                            plsc.addupdate_scatter(band_v, [idx], wgt, mask=m_lo_rest)
                        elif m == 7 and tap == 2:
                            plsc.addupdate_scatter(band_v, [idx], wgt, mask=m_hi)
                            plsc.addupdate_scatter(band_v, [idx], wgt, mask=m_hi_rest)
                        else:
                            plsc.addupdate_scatter(band_v, [idx], wgt)

    pltpu.sync_copy(band_v, out_hbm.at[c, s])


def _scatter_bands(cls):
    b = cls.shape[0]
    mesh = plsc.VectorSubcoreMesh(
        core_axis_name="c", subcore_axis_name="s",
        num_cores=b, num_subcores=NSUB)
    fn = functools.partial(
        pl.kernel,
        out_type=jax.ShapeDtypeStruct((b, NSUB, BAND), jnp.float32),
        mesh=mesh,
        scratch_types=[
            pltpu.VMEM((ROWS_PER_TILE * HF,), jnp.int32),
            pltpu.VMEM((BAND,), jnp.float32),
        ],
        compiler_params=pltpu.CompilerParams(needs_layout_passes=False,
                                             disable_bounds_checks=True),
    )(_sc_body)
    return fn(cls)


def _proto_body(bands_ref, feats_ref, out_ref, w_ref):
    b = pl.program_id(0)
    nb = pl.num_programs(0)
    w_ref[...] = jnp.zeros((CPAD, HC * HC), jnp.float32)
    for s in range(NSUB):
        lo = (8 * s - 1) * HC
        a = max(lo, 0)
        e = min(lo + BROWS * HC, HC * HC)
        w_ref[:, a:e] = w_ref[:, a:e] + bands_ref[0, s][:, a - lo:e - lo]
    wf = w_ref[0:NCLS, :]
    sums = jnp.sum(wf, axis=1)
    num = lax.dot_general(wf, feats_ref[0], (((1,), (1,)), ((), ())),
                          preferred_element_type=jnp.float32)
    contrib = num / (sums + 1e-6)[:, None] * (1.0 / nb)

    @pl.when(b == 0)
    def _init():
        out_ref[...] = contrib

    @pl.when(b != 0)
    def _acc():
        out_ref[...] = out_ref[...] + contrib


def _prototypes(bands, feats2d):
    b, c = feats2d.shape[0], feats2d.shape[1]
    return pl.pallas_call(
        _proto_body,
        grid=(b,),
        in_specs=[
            pl.BlockSpec((1, NSUB, CPAD, BROWS * HC), lambda i: (i, 0, 0, 0)),
            pl.BlockSpec((1, c, HC * HC), lambda i: (i, 0, 0)),
        ],
        out_specs=pl.BlockSpec((NCLS, c), lambda i: (0, 0)),
        out_shape=jax.ShapeDtypeStruct((NCLS, c), jnp.float32),
        scratch_shapes=[pltpu.VMEM((CPAD, HC * HC), jnp.float32)],
    )(bands, feats2d)


def kernel(feats, preds, masks):
    b, c = feats.shape[0], feats.shape[1]
    masks = masks.astype(jnp.int32)
    cls = _compute_cls(preds, masks)
    bands = _scatter_bands(cls.reshape(b, HF * HF))
    bands = bands.reshape(b, NSUB, CPAD, BROWS * HC)
    feats2d = feats.reshape(b, c, HC * HC)
    return _prototypes(bands, feats2d)


# speedup vs baseline: 1.1462x; 1.0564x over previous
import functools

import jax
import jax.numpy as jnp
from jax import lax
from jax.experimental import pallas as pl
from jax.experimental.pallas import tpu as pltpu
from jax.experimental.pallas import tpu_sc as plsc

NCLS = 19
CPAD = 20
HC = 128
HF = 512
BROWS = 10
BAND = CPAD * BROWS * HC
NSUB = 16
ROWS_PER_TILE = HF // NSUB


def _cls_body(preds_ref, masks_ref, cls_ref):
    best = preds_ref[0, 0]
    besti = jnp.zeros(best.shape, jnp.int32)
    for k in range(1, NCLS):
        pk = preds_ref[0, k]
        m = pk > best
        best = jnp.where(m, pk, best)
        besti = jnp.where(m, k, besti)
    mask = masks_ref[0]
    cls_ref[0] = jnp.where(besti == mask, mask, NCLS).astype(jnp.int32)


def _compute_cls(preds, masks):
    b = preds.shape[0]
    rb = 128
    grid = (b, HF // rb)
    return pl.pallas_call(
        _cls_body,
        grid=grid,
        in_specs=[
            pl.BlockSpec((1, NCLS, rb, HF), lambda i, j: (i, 0, j, 0)),
            pl.BlockSpec((1, rb, HF), lambda i, j: (i, j, 0)),
        ],
        out_specs=pl.BlockSpec((1, rb, HF), lambda i, j: (i, j, 0)),
        out_shape=jax.ShapeDtypeStruct((b, HF, HF), jnp.int32),
    )(preds, masks)


def _sc_body(cls_hbm, out_hbm, cls_v, band_v):
    c = lax.axis_index("c")
    s = lax.axis_index("s")
    npix = ROWS_PER_TILE * HF
    pltpu.sync_copy(cls_hbm.at[c, pl.ds(s * npix, npix)], cls_v)

    zero16 = jnp.zeros((16,), jnp.float32)

    @plsc.parallel_loop(0, BAND // 256)
    def _zero(i):
        for u in range(16):
            band_v[pl.ds(i * 256 + u * 16, 16)] = zero16

    lane = lax.iota(jnp.int32, 16)
    lane4 = lane * 4
    band_base_row = 8 * s - 1
    wtab = (0.375, 0.125, 0.875, 0.625)

    colm = []
    for m in range(8):
        c0 = lane + (16 * m - 1)
        if m == 0:
            c0 = jnp.maximum(c0, 0)
        c1 = lane + 16 * m
        c2 = lane + (16 * m + 1)
        if m == 7:
            c2 = jnp.minimum(c2, HC - 1)
        colm.append((c0, c1, c2))
    m_lo = lane == 0
    m_lo_rest = lane != 0
    m_hi = lane == 15
    m_hi_rest = lane != 15

    @plsc.parallel_loop(0, ROWS_PER_TILE, unroll=4)
    def _row(i):
        ih = s * ROWS_PER_TILE + i
        jh = ih // 4
        rh = ih % 4
        jha = jnp.clip(jh + jnp.where(rh < 2, -1, 0), 0, HC - 1)
        jhb = jnp.clip(jh + jnp.where(rh < 2, 0, 1), 0, HC - 1)
        wha = jnp.where(rh == 0, 0.375,
              jnp.where(rh == 1, 0.125,
              jnp.where(rh == 2, 0.875, 0.625)))
        whb = 1.0 - wha
        rowa = jnp.full((16,), (jha - band_base_row) * HC, jnp.int32)
        rowb = jnp.full((16,), (jhb - band_base_row) * HC, jnp.int32)
        wa = [jnp.full((16,), wha * x, jnp.float32) for x in wtab]
        wb = [jnp.full((16,), whb * x, jnp.float32) for x in wtab]
        pix0 = lane4 + i * HF
        for m in range(8):
            c0, c1, c2 = colm[m]
            ra = (rowa + c0, rowa + c1, rowa + c2)
            rb = (rowb + c0, rowb + c1, rowb + c2)
            cls4 = [plsc.load_gather(cls_v, [pix0 + (64 * m + r)])
                    for r in range(4)]
            base4 = [cv * (BROWS * HC) for cv in cls4]
            for r in range(4):
                la = 0 if r < 2 else 1
                base = base4[r]
                for rowv, wv in ((ra, wa), (rb, wb)):
                    for tap, wgt in ((la, wv[r]), (la + 1, wv[3 - r])):
                        idx = base + rowv[tap]
                        if m == 0 and tap == 0:
                            plsc.addupdate_scatter(band_v, [idx], wgt, mask=m_lo)
                            plsc.addupdate_scatter(band_v, [idx], wgt, mask=m_lo_rest)
                        elif m == 7 and tap == 2:
                            plsc.addupdate_scatter(band_v, [idx], wgt, mask=m_hi)
                            plsc.addupdate_scatter(band_v, [idx], wgt, mask=m_hi_rest)
                        else:
                            plsc.addupdate_scatter(band_v, [idx], wgt)

    pltpu.sync_copy(band_v, out_hbm.at[c, s])


def _scatter_bands(cls):
    b = cls.shape[0]
    mesh = plsc.VectorSubcoreMesh(
        core_axis_name="c", subcore_axis_name="s",
        num_cores=b, num_subcores=NSUB)
    fn = functools.partial(
        pl.kernel,
        out_type=jax.ShapeDtypeStruct((b, NSUB, BAND), jnp.float32),
        mesh=mesh,
        scratch_types=[
            pltpu.VMEM((ROWS_PER_TILE * HF,), jnp.int32),
            pltpu.VMEM((BAND,), jnp.float32),
        ],
        compiler_params=pltpu.CompilerParams(needs_layout_passes=False,
                                             disable_bounds_checks=True),
    )(_sc_body)
    return fn(cls)


def _proto_body(bands_ref, feats_ref, out_ref, w_ref):
    b = pl.program_id(0)
    nb = pl.num_programs(0)
    w_ref[...] = jnp.zeros((CPAD, HC * HC), jnp.float32)
    for s in range(NSUB):
        lo = (8 * s - 1) * HC
        a = max(lo, 0)
        e = min(lo + BROWS * HC, HC * HC)
        w_ref[:, a:e] = w_ref[:, a:e] + bands_ref[0, s][:, a - lo:e - lo]
    wf = w_ref[0:NCLS, :]
    sums = jnp.sum(wf, axis=1)
    num = lax.dot_general(wf, feats_ref[0], (((1,), (1,)), ((), ())),
                          preferred_element_type=jnp.float32)
    contrib = num / (sums + 1e-6)[:, None] * (1.0 / nb)

    @pl.when(b == 0)
    def _init():
        out_ref[...] = contrib

    @pl.when(b != 0)
    def _acc():
        out_ref[...] = out_ref[...] + contrib


def _prototypes(bands, feats2d):
    b, c = feats2d.shape[0], feats2d.shape[1]
    return pl.pallas_call(
        _proto_body,
        grid=(b,),
        in_specs=[
            pl.BlockSpec((1, NSUB, CPAD, BROWS * HC), lambda i: (i, 0, 0, 0)),
            pl.BlockSpec((1, c, HC * HC), lambda i: (i, 0, 0)),
        ],
        out_specs=pl.BlockSpec((NCLS, c), lambda i: (0, 0)),
        out_shape=jax.ShapeDtypeStruct((NCLS, c), jnp.float32),
        scratch_shapes=[pltpu.VMEM((CPAD, HC * HC), jnp.float32)],
    )(bands, feats2d)


def kernel(feats, preds, masks):
    b, c = feats.shape[0], feats.shape[1]
    masks = masks.astype(jnp.int32)
    cls = _compute_cls(preds, masks)
    bands = _scatter_bands(cls.reshape(b, HF * HF))
    bands = bands.reshape(b, NSUB, CPAD, BROWS * HC)
    feats2d = feats.reshape(b, c, HC * HC)
    return _prototypes(bands, feats2d)
